# baseline (speedup 1.0000x reference)
import functools

import jax
import jax.numpy as jnp
from jax.experimental import pallas as pl
from jax.experimental.pallas import tpu as pltpu

_S = 8
_U = 4


def _mish(x):
    e = jnp.exp(jnp.minimum(x, 20.0))
    g = e * (e + 2.0)
    return x * (g / (g + 2.0))


def _flat_geom(Ho, Wo):
    Wf = Wo + 8
    A = ((2 * Wf + 1) // 8) * 8
    dmax = (Ho + 1) * Wf + Wo
    M = -(-(dmax - A + 1) // 8) * 8
    R = -(-(A + Wf + 1 + M) // 8) * 8
    return Wf, A, M, R


def _impala_kernel(c4_ref, we_ref, wo_ref, bc_ref, wr_ref, br_ref,
                   out_ref, cop_ref, xm_ref, fx_ref, fh_ref, sh_ref,
                   *, H, W, Cin, C):
    Ho, Wo = H // 2, W // 2
    Lo = _S * C
    Wf, A, M, R = _flat_geom(Ho, Wo)
    Ms = H * Wf
    zv = jnp.zeros((8, Lo), jnp.float32)

    def stem_pool(u):
        acce = acco = None
        for dy in range(3):
            op = c4_ref[u, dy * Wf:dy * Wf + Ms, :]
            de = jnp.dot(op, we_ref[dy * Lo:(dy + 1) * Lo, :],
                         preferred_element_type=jnp.float32)
            do = jnp.dot(op, wo_ref[dy * Lo:(dy + 1) * Lo, :],
                         preferred_element_type=jnp.float32)
            acce = de if acce is None else acce + de
            acco = do if acco is None else acco + do
        bce = bc_ref[...]
        ce = _mish((acce + bce).reshape(H, Wf, Lo)[:, 0:Wo, :])
        co = _mish((acco + bce).reshape(H, Wf, Lo)[:, 0:Wo, :])

        cop_ref[u, :, 0:1, :] = jnp.full((H, 1, Lo), -jnp.inf, jnp.float32)
        cop_ref[u, :, 1:Wo + 1, :] = co
        xm_ref[u, 0:1] = jnp.full((1, Wo, Lo), -jnp.inf, jnp.float32)
        xm_ref[u, H + 1:H + 2] = jnp.full((1, Wo, Lo), -jnp.inf, jnp.float32)
        xm_ref[u, 1:H + 1] = jnp.maximum(
            ce, jnp.maximum(cop_ref[u, :, 1:Wo + 1, :], cop_ref[u, :, 0:Wo, :]))
        ym = jnp.maximum(xm_ref[u, 0:H], jnp.maximum(xm_ref[u, 1:H + 1],
                                                     xm_ref[u, 2:H + 2]))
        m = ym.reshape(Ho, 2, Wo, Lo)[:, 0]

        fx_ref[u, 0:A, :] = jnp.zeros((A, Lo), jnp.float32)
        fx_ref[u, A + M:R, :] = jnp.zeros((R - A - M, Lo), jnp.float32)
        fh_ref[u, 0:A, :] = jnp.zeros((A, Lo), jnp.float32)
        fh_ref[u, A + M:R, :] = jnp.zeros((R - A - M, Lo), jnp.float32)
        mp = jnp.concatenate(
            [jnp.zeros((Ho, 1, Lo), jnp.float32), m,
             jnp.zeros((Ho, Wf - Wo - 1, Lo), jnp.float32)], axis=1)
        fx_ref[u, 2 * Wf:(Ho + 2) * Wf, :] = mp.reshape(Ho * Wf, Lo)
        if 2 * Wf + 1 > A:
            fx_ref[u, A:2 * Wf + 1, :] = (
                jnp.zeros((2 * Wf + 1 - A, Lo), jnp.float32))
        if A + M > (Ho + 2) * Wf:
            fx_ref[u, (Ho + 2) * Wf:A + M, :] = (
                jnp.zeros((A + M - (Ho + 2) * Wf, Lo), jnp.float32))

    def zero_pads(f_ref, u):
        if 2 * Wf + 1 > A:
            f_ref[u, A:2 * Wf + 1, :] = (
                jnp.zeros((2 * Wf + 1 - A, Lo), jnp.float32))
        for j in range(2, Ho + 2):
            f_ref[u, j * Wf + Wo + 1:j * Wf + Wf + 1, :] = zv

    def conv(u, src_ref, widx, dst_ref, res_ref):
        sh_ref[u, ...] = src_ref[u, A - Wf - 1:A + Wf - 1 + M, :]
        acc = None
        for dy in range(3):
            for dx in (0, 1):
                t = dy * 3 + dx
                w_t = wr_ref[(widx * 9 + t) * Lo:(widx * 9 + t + 1) * Lo, :]
                if dx == 1:
                    op = src_ref[u, A + (dy - 1) * Wf:A + (dy - 1) * Wf + M, :]
                else:
                    op = sh_ref[u, dy * Wf:dy * Wf + M, :]
                d = jnp.dot(op, w_t, preferred_element_type=jnp.float32)
                acc = d if acc is None else acc + d
        sh_ref[u, ...] = src_ref[u, A - Wf + 1:A + Wf + 1 + M, :]
        for dy in range(3):
            t = dy * 3 + 2
            w_t = wr_ref[(widx * 9 + t) * Lo:(widx * 9 + t + 1) * Lo, :]
            d = jnp.dot(sh_ref[u, dy * Wf:dy * Wf + M, :], w_t,
                        preferred_element_type=jnp.float32)
            acc = acc + d
        y = _mish(acc + br_ref[widx:widx + 1, :])
        if res_ref is not None:
            y = y + res_ref[u, A:A + M, :]
        dst_ref[u, A:A + M, :] = y
        zero_pads(dst_ref, u)

    for u in range(_U):
        stem_pool(u)
    for u in range(_U):
        conv(u, fx_ref, 0, fh_ref, None)
    for u in range(_U):
        conv(u, fh_ref, 1, fx_ref, fx_ref)
    for u in range(_U):
        conv(u, fx_ref, 2, fh_ref, None)
    for u in range(_U):
        conv(u, fh_ref, 3, fx_ref, fx_ref)
    for u in range(_U):
        yf = fx_ref[u, 2 * Wf:(Ho + 2) * Wf, :].reshape(Ho, Wf, Lo)
        out_ref[u, ...] = yf[:, 1:Wo + 1, :]


def _block_diag_w(w, ci, co):
    eye = jnp.eye(_S, dtype=w.dtype)
    wb = jnp.einsum('su,co->scuo', eye, w)
    return wb.reshape(_S * ci, _S * co)


def kernel(x, conv_w, conv_b, res1_w1, res1_b1, res1_w2, res1_b2,
           res2_w1, res2_b1, res2_w2, res2_b2):
    n, cin, h, w = x.shape
    cout = conv_w.shape[-1]
    ho, wo = h // 2, w // 2
    g = n // _S
    wf, fa, fm, fr = _flat_geom(ho, wo)
    li, lo = _S * cin, _S * cout

    xg = jnp.transpose(x.reshape(g, _S, cin, h, w), (0, 3, 4, 1, 2))
    xg = xg.reshape(g, h, w, li)
    xp = jnp.pad(xg, ((0, 0), (1, 1), (1, 1), (0, 0)))
    a = xp[:, :, 0::2, :]
    b = xp[:, :, 1::2, :]
    zcol = jnp.zeros((g, h + 2, 1, li), jnp.float32)
    a1 = jnp.concatenate([a[:, :, 1:, :], zcol], axis=2)
    b1 = jnp.concatenate([b[:, :, 1:, :], zcol], axis=2)
    c4 = jnp.concatenate([a, b, a1, b1], axis=-1)
    c4 = jnp.pad(c4, ((0, 0), (0, 0), (0, wf - wo - 1), (0, 0)))
    c4 = c4.reshape(g, (h + 2) * wf, 4 * li)

    wc9 = conv_w.reshape(3, 3, cin, cout)
    zblk = jnp.zeros((li, lo), jnp.float32)
    we = jnp.concatenate([
        jnp.concatenate([_block_diag_w(wc9[dy, 0], cin, cout),
                         _block_diag_w(wc9[dy, 1], cin, cout),
                         _block_diag_w(wc9[dy, 2], cin, cout), zblk], axis=0)
        for dy in range(3)], axis=0)
    wod = jnp.concatenate([
        jnp.concatenate([zblk, _block_diag_w(wc9[dy, 0], cin, cout),
                         _block_diag_w(wc9[dy, 1], cin, cout),
                         _block_diag_w(wc9[dy, 2], cin, cout)], axis=0)
        for dy in range(3)], axis=0)
    wr = jnp.concatenate(
        [jnp.concatenate([_block_diag_w(m.reshape(9, cout, cout)[t],
                                        cout, cout) for t in range(9)], axis=0)
         for m in (res1_w1, res1_w2, res2_w1, res2_w2)], axis=0)
    bc = jnp.tile(conv_b.reshape(1, cout), (1, _S))
    br = jnp.concatenate(
        [jnp.tile(v.reshape(1, cout), (1, _S))
         for v in (res1_b1, res1_b2, res2_b1, res2_b2)], axis=0)

    kern = functools.partial(_impala_kernel, H=h, W=w, Cin=cin, C=cout)
    out = pl.pallas_call(
        kern,
        grid=(g // _U,),
        in_specs=[
            pl.BlockSpec((_U, (h + 2) * wf, 4 * li), lambda i: (i, 0, 0)),
            pl.BlockSpec((3 * 4 * li, lo), lambda i: (0, 0)),
            pl.BlockSpec((3 * 4 * li, lo), lambda i: (0, 0)),
            pl.BlockSpec((1, lo), lambda i: (0, 0)),
            pl.BlockSpec((4 * 9 * lo, lo), lambda i: (0, 0)),
            pl.BlockSpec((4, lo), lambda i: (0, 0)),
        ],
        out_specs=pl.BlockSpec((_U, ho, wo, lo), lambda i: (i, 0, 0, 0)),
        out_shape=jax.ShapeDtypeStruct((g, ho, wo, lo), jnp.float32),
        scratch_shapes=[
            pltpu.VMEM((_U, h, wo + 1, lo), jnp.float32),
            pltpu.VMEM((_U, h + 2, wo, lo), jnp.float32),
            pltpu.VMEM((_U, fr, lo), jnp.float32),
            pltpu.VMEM((_U, fr, lo), jnp.float32),
            pltpu.VMEM((_U, 2 * wf + fm, lo), jnp.float32),
        ],
        compiler_params=pltpu.CompilerParams(
            dimension_semantics=("parallel",),
            vmem_limit_bytes=100 * 1024 * 1024),
        cost_estimate=pl.CostEstimate(
            flops=2 * n * (h * w * 9 * cin * cout
                           + 4 * ho * wo * 9 * cout * cout),
            transcendentals=n * (h * w + 4 * ho * wo) * cout,
            bytes_accessed=4 * (g * (h + 2) * wf * 4 * li
                                + g * ho * wo * lo),
        ),
    )(c4, we, wod, bc, wr, br)
    out = out.reshape(g, ho, wo, _S, cout)
    return jnp.transpose(out, (0, 3, 4, 1, 2)).reshape(n, cout, ho, wo)


# speedup vs baseline: 1.3337x; 1.3337x over previous
import functools

import jax
import jax.numpy as jnp
from jax.experimental import pallas as pl
from jax.experimental.pallas import tpu as pltpu

_S = 8
_U = 4


def _mish(x):
    e = jnp.exp(jnp.minimum(x, 20.0))
    g = e * (e + 2.0)
    return x * (g / (g + 2.0))


def _flat_geom(Ho, Wo):
    Wf = Wo + 8
    A = ((2 * Wf + 1) // 8) * 8
    dmax = (Ho + 1) * Wf + Wo
    M = -(-(dmax - A + 1) // 8) * 8
    R = -(-(A + Wf + 1 + M) // 8) * 8
    return Wf, A, M, R


def _impala_kernel(xe_ref, xo_ref, i32_ref, i128_ref,
                   we_ref, wo_ref, bc_ref, wr_ref, br_ref,
                   out_ref, c4_ref, cop_ref, xm_ref, fx_ref, fh_ref, sh_ref,
                   *, H, W, Cin, C):
    Ho, Wo = H // 2, W // 2
    Li, Lo = _S * Cin, _S * C
    Wf, A, M, R = _flat_geom(Ho, Wo)
    Ms = H * Wf
    zv = jnp.zeros((8, Lo), jnp.float32)
    tdims = (((0,), (0,)), ((), ()))

    def pack_input(u):
        bt = jax.lax.dot_general(xe_ref[u], i32_ref[...], tdims,
                                 preferred_element_type=jnp.float32)
        at = jax.lax.dot_general(xo_ref[u], i32_ref[...], tdims,
                                 preferred_element_type=jnp.float32)
        at3 = at.reshape(H, Wo, Li)
        bt3 = bt.reshape(H, Wo, Li)
        z1 = jnp.zeros((H, 1, Li), jnp.float32)
        z7 = jnp.zeros((H, Wf - Wo - 1, Li), jnp.float32)
        z8 = jnp.zeros((H, Wf - Wo, Li), jnp.float32)
        z9 = jnp.zeros((H, Wf - Wo + 1, Li), jnp.float32)
        v = jnp.concatenate([
            jnp.concatenate([z1, at3, z7], axis=1),
            jnp.concatenate([bt3, z8], axis=1),
            jnp.concatenate([at3, z8], axis=1),
            jnp.concatenate([bt3[:, 1:Wo, :], z9], axis=1),
        ], axis=2)
        c4_ref[u, 0:Wf, :] = jnp.zeros((Wf, 4 * Li), jnp.float32)
        c4_ref[u, Wf:Wf + H * Wf, :] = v.reshape(H * Wf, 4 * Li)
        c4_ref[u, (H + 1) * Wf:(H + 2) * Wf, :] = (
            jnp.zeros((Wf, 4 * Li), jnp.float32))

    def stem_pool(u):
        acce = acco = None
        for dy in range(3):
            op = c4_ref[u, dy * Wf:dy * Wf + Ms, :]
            de = jnp.dot(op, we_ref[dy * Lo:(dy + 1) * Lo, :],
                         preferred_element_type=jnp.float32)
            do = jnp.dot(op, wo_ref[dy * Lo:(dy + 1) * Lo, :],
                         preferred_element_type=jnp.float32)
            acce = de if acce is None else acce + de
            acco = do if acco is None else acco + do
        bce = bc_ref[...]
        ce = _mish((acce + bce).reshape(H, Wf, Lo)[:, 0:Wo, :])
        co = _mish((acco + bce).reshape(H, Wf, Lo)[:, 0:Wo, :])

        cop_ref[u, :, 0:1, :] = jnp.full((H, 1, Lo), -jnp.inf, jnp.float32)
        cop_ref[u, :, 1:Wo + 1, :] = co
        xm_ref[u, 0:1] = jnp.full((1, Wo, Lo), -jnp.inf, jnp.float32)
        xm_ref[u, H + 1:H + 2] = jnp.full((1, Wo, Lo), -jnp.inf, jnp.float32)
        xm_ref[u, 1:H + 1] = jnp.maximum(
            ce, jnp.maximum(cop_ref[u, :, 1:Wo + 1, :], cop_ref[u, :, 0:Wo, :]))
        ym = jnp.maximum(xm_ref[u, 0:H], jnp.maximum(xm_ref[u, 1:H + 1],
                                                     xm_ref[u, 2:H + 2]))
        m = ym.reshape(Ho, 2, Wo, Lo)[:, 0]

        fx_ref[u, 0:A, :] = jnp.zeros((A, Lo), jnp.float32)
        fx_ref[u, A + M:R, :] = jnp.zeros((R - A - M, Lo), jnp.float32)
        fh_ref[u, 0:A, :] = jnp.zeros((A, Lo), jnp.float32)
        fh_ref[u, A + M:R, :] = jnp.zeros((R - A - M, Lo), jnp.float32)
        mp = jnp.concatenate(
            [jnp.zeros((Ho, 1, Lo), jnp.float32), m,
             jnp.zeros((Ho, Wf - Wo - 1, Lo), jnp.float32)], axis=1)
        fx_ref[u, 2 * Wf:(Ho + 2) * Wf, :] = mp.reshape(Ho * Wf, Lo)
        if 2 * Wf + 1 > A:
            fx_ref[u, A:2 * Wf + 1, :] = (
                jnp.zeros((2 * Wf + 1 - A, Lo), jnp.float32))
        if A + M > (Ho + 2) * Wf:
            fx_ref[u, (Ho + 2) * Wf:A + M, :] = (
                jnp.zeros((A + M - (Ho + 2) * Wf, Lo), jnp.float32))

    def zero_pads(f_ref, u):
        if 2 * Wf + 1 > A:
            f_ref[u, A:2 * Wf + 1, :] = (
                jnp.zeros((2 * Wf + 1 - A, Lo), jnp.float32))
        for j in range(2, Ho + 2):
            f_ref[u, j * Wf + Wo + 1:j * Wf + Wf + 1, :] = zv

    def conv(u, src_ref, widx, dst_ref, res_ref):
        sh_ref[u, ...] = src_ref[u, A - Wf - 1:A + Wf - 1 + M, :]
        acc = None
        for dy in range(3):
            for dx in (0, 1):
                t = dy * 3 + dx
                w_t = wr_ref[(widx * 9 + t) * Lo:(widx * 9 + t + 1) * Lo, :]
                if dx == 1:
                    op = src_ref[u, A + (dy - 1) * Wf:A + (dy - 1) * Wf + M, :]
                else:
                    op = sh_ref[u, dy * Wf:dy * Wf + M, :]
                d = jnp.dot(op, w_t, preferred_element_type=jnp.float32)
                acc = d if acc is None else acc + d
        sh_ref[u, ...] = src_ref[u, A - Wf + 1:A + Wf + 1 + M, :]
        for dy in range(3):
            t = dy * 3 + 2
            w_t = wr_ref[(widx * 9 + t) * Lo:(widx * 9 + t + 1) * Lo, :]
            d = jnp.dot(sh_ref[u, dy * Wf:dy * Wf + M, :], w_t,
                        preferred_element_type=jnp.float32)
            acc = acc + d
        y = _mish(acc + br_ref[widx:widx + 1, :])
        if res_ref is not None:
            y = y + res_ref[u, A:A + M, :]
        dst_ref[u, A:A + M, :] = y
        zero_pads(dst_ref, u)

    for u in range(_U):
        pack_input(u)
    for u in range(_U):
        stem_pool(u)
    for u in range(_U):
        conv(u, fx_ref, 0, fh_ref, None)
    for u in range(_U):
        conv(u, fh_ref, 1, fx_ref, fx_ref)
    for u in range(_U):
        conv(u, fx_ref, 2, fh_ref, None)
    for u in range(_U):
        conv(u, fh_ref, 3, fx_ref, fx_ref)
    for u in range(_U):
        yf = fx_ref[u, 2 * Wf:(Ho + 2) * Wf, :].reshape(Ho, Wf, Lo)
        yv = yf[:, 1:Wo + 1, :].reshape(Ho * Wo, Lo)
        out_ref[u, ...] = jax.lax.dot_general(
            i128_ref[...], yv, (((1,), (1,)), ((), ())),
            preferred_element_type=jnp.float32)


def _block_diag_w(w, ci, co):
    eye = jnp.eye(_S, dtype=w.dtype)
    wb = jnp.einsum('su,co->scuo', eye, w)
    return wb.reshape(_S * ci, _S * co)


def kernel(x, conv_w, conv_b, res1_w1, res1_b1, res1_w2, res1_b2,
           res2_w1, res2_b1, res2_w2, res2_b2):
    n, cin, h, w = x.shape
    cout = conv_w.shape[-1]
    ho, wo = h // 2, w // 2
    g = n // _S
    wf, fa, fm, fr = _flat_geom(ho, wo)
    li, lo = _S * cin, _S * cout

    xv = x.reshape(g, li, h, w)
    xe = xv[:, :, :, 0::2].reshape(g, li, h * wo)
    xo = xv[:, :, :, 1::2].reshape(g, li, h * wo)
    i32 = jnp.eye(li, dtype=jnp.float32)
    i128 = jnp.eye(lo, dtype=jnp.float32)

    wc9 = conv_w.reshape(3, 3, cin, cout)
    zblk = jnp.zeros((li, lo), jnp.float32)
    we = jnp.concatenate([
        jnp.concatenate([_block_diag_w(wc9[dy, 0], cin, cout),
                         _block_diag_w(wc9[dy, 1], cin, cout),
                         _block_diag_w(wc9[dy, 2], cin, cout), zblk], axis=0)
        for dy in range(3)], axis=0)
    wod = jnp.concatenate([
        jnp.concatenate([zblk, _block_diag_w(wc9[dy, 0], cin, cout),
                         _block_diag_w(wc9[dy, 1], cin, cout),
                         _block_diag_w(wc9[dy, 2], cin, cout)], axis=0)
        for dy in range(3)], axis=0)
    wr = jnp.concatenate(
        [jnp.concatenate([_block_diag_w(m.reshape(9, cout, cout)[t],
                                        cout, cout) for t in range(9)], axis=0)
         for m in (res1_w1, res1_w2, res2_w1, res2_w2)], axis=0)
    bc = jnp.tile(conv_b.reshape(1, cout), (1, _S))
    br = jnp.concatenate(
        [jnp.tile(v.reshape(1, cout), (1, _S))
         for v in (res1_b1, res1_b2, res2_b1, res2_b2)], axis=0)

    kern = functools.partial(_impala_kernel, H=h, W=w, Cin=cin, C=cout)
    out = pl.pallas_call(
        kern,
        grid=(g // _U,),
        in_specs=[
            pl.BlockSpec((_U, li, h * wo), lambda i: (i, 0, 0)),
            pl.BlockSpec((_U, li, h * wo), lambda i: (i, 0, 0)),
            pl.BlockSpec((li, li), lambda i: (0, 0)),
            pl.BlockSpec((lo, lo), lambda i: (0, 0)),
            pl.BlockSpec((3 * 4 * li, lo), lambda i: (0, 0)),
            pl.BlockSpec((3 * 4 * li, lo), lambda i: (0, 0)),
            pl.BlockSpec((1, lo), lambda i: (0, 0)),
            pl.BlockSpec((4 * 9 * lo, lo), lambda i: (0, 0)),
            pl.BlockSpec((4, lo), lambda i: (0, 0)),
        ],
        out_specs=pl.BlockSpec((_U, lo, ho * wo), lambda i: (i, 0, 0)),
        out_shape=jax.ShapeDtypeStruct((g, lo, ho * wo), jnp.float32),
        scratch_shapes=[
            pltpu.VMEM((_U, (h + 2) * wf, 4 * li), jnp.float32),
            pltpu.VMEM((_U, h, wo + 1, lo), jnp.float32),
            pltpu.VMEM((_U, h + 2, wo, lo), jnp.float32),
            pltpu.VMEM((_U, fr, lo), jnp.float32),
            pltpu.VMEM((_U, fr, lo), jnp.float32),
            pltpu.VMEM((_U, 2 * wf + fm, lo), jnp.float32),
        ],
        compiler_params=pltpu.CompilerParams(
            dimension_semantics=("parallel",),
            vmem_limit_bytes=100 * 1024 * 1024),
        cost_estimate=pl.CostEstimate(
            flops=2 * n * (h * w * 9 * cin * cout
                           + 4 * ho * wo * 9 * cout * cout),
            transcendentals=n * (h * w + 4 * ho * wo) * cout,
            bytes_accessed=4 * (2 * g * li * h * wo + g * lo * ho * wo),
        ),
    )(xe, xo, i32, i128, we, wod, bc, wr, br)
    return out.reshape(n, cout, ho, wo)


# speedup vs baseline: 3.1779x; 2.3827x over previous
import functools

import jax
import jax.numpy as jnp
from jax.experimental import pallas as pl
from jax.experimental.pallas import tpu as pltpu

_S = 8
_U = 4


def _mish(x):
    e = jnp.exp(jnp.minimum(x, 20.0))
    g = e * (e + 2.0)
    return x * (g / (g + 2.0))


def _flat_geom(Ho, Wo):
    Wf = Wo + 8
    A = ((2 * Wf + 1) // 8) * 8
    dmax = (Ho + 1) * Wf + Wo
    M = -(-(dmax - A + 1) // 8) * 8
    R = -(-(A + Wf + 1 + M) // 8) * 8
    return Wf, A, M, R


def _impala_kernel(x_ref, i32_ref, i128_ref,
                   we_ref, wo_ref, bc_ref, wr_ref, br_ref,
                   out_ref, c4_ref, cop_ref, xm_ref, fx_ref, fh_ref, sh_ref,
                   *, H, W, Cin, C):
    Ho, Wo = H // 2, W // 2
    Li, Lo = _S * Cin, _S * C
    Wf, A, M, R = _flat_geom(Ho, Wo)
    Ms = H * Wf
    zv = jnp.zeros((8, Lo), jnp.float32)
    tdims = (((0,), (0,)), ((), ()))

    def pack_input(u):
        x4 = x_ref[u]
        eidx = jax.lax.broadcasted_iota(jnp.int32, (Li, H, Wo), 2) * 2
        xe = jnp.take_along_axis(x4, eidx, axis=2).reshape(Li, H * Wo)
        xo = jnp.take_along_axis(x4, eidx + 1, axis=2).reshape(Li, H * Wo)
        bt = jax.lax.dot_general(xe, i32_ref[...], tdims,
                                 preferred_element_type=jnp.float32)
        at = jax.lax.dot_general(xo, i32_ref[...], tdims,
                                 preferred_element_type=jnp.float32)
        at3 = at.reshape(H, Wo, Li)
        bt3 = bt.reshape(H, Wo, Li)
        z1 = jnp.zeros((H, 1, Li), jnp.float32)
        z7 = jnp.zeros((H, Wf - Wo - 1, Li), jnp.float32)
        z8 = jnp.zeros((H, Wf - Wo, Li), jnp.float32)
        z9 = jnp.zeros((H, Wf - Wo + 1, Li), jnp.float32)
        v = jnp.concatenate([
            jnp.concatenate([z1, at3, z7], axis=1),
            jnp.concatenate([bt3, z8], axis=1),
            jnp.concatenate([at3, z8], axis=1),
            jnp.concatenate([bt3[:, 1:Wo, :], z9], axis=1),
        ], axis=2)
        c4_ref[u, 0:Wf, :] = jnp.zeros((Wf, 4 * Li), jnp.float32)
        c4_ref[u, Wf:Wf + H * Wf, :] = v.reshape(H * Wf, 4 * Li)
        c4_ref[u, (H + 1) * Wf:(H + 2) * Wf, :] = (
            jnp.zeros((Wf, 4 * Li), jnp.float32))

    def stem_pool(u):
        acce = acco = None
        for dy in range(3):
            op = c4_ref[u, dy * Wf:dy * Wf + Ms, :]
            de = jnp.dot(op, we_ref[dy * Lo:(dy + 1) * Lo, :],
                         preferred_element_type=jnp.float32)
            do = jnp.dot(op, wo_ref[dy * Lo:(dy + 1) * Lo, :],
                         preferred_element_type=jnp.float32)
            acce = de if acce is None else acce + de
            acco = do if acco is None else acco + do
        bce = bc_ref[...]
        ce = _mish((acce + bce).reshape(H, Wf, Lo)[:, 0:Wo, :])
        co = _mish((acco + bce).reshape(H, Wf, Lo)[:, 0:Wo, :])

        cop_ref[u, :, 0:1, :] = jnp.full((H, 1, Lo), -jnp.inf, jnp.float32)
        cop_ref[u, :, 1:Wo + 1, :] = co
        xm_ref[u, 0:1] = jnp.full((1, Wo, Lo), -jnp.inf, jnp.float32)
        xm_ref[u, H + 1:H + 2] = jnp.full((1, Wo, Lo), -jnp.inf, jnp.float32)
        xm_ref[u, 1:H + 1] = jnp.maximum(
            ce, jnp.maximum(cop_ref[u, :, 1:Wo + 1, :], cop_ref[u, :, 0:Wo, :]))
        ym = jnp.maximum(xm_ref[u, 0:H], jnp.maximum(xm_ref[u, 1:H + 1],
                                                     xm_ref[u, 2:H + 2]))
        m = ym.reshape(Ho, 2, Wo, Lo)[:, 0]

        fx_ref[u, 0:A, :] = jnp.zeros((A, Lo), jnp.float32)
        fx_ref[u, A + M:R, :] = jnp.zeros((R - A - M, Lo), jnp.float32)
        fh_ref[u, 0:A, :] = jnp.zeros((A, Lo), jnp.float32)
        fh_ref[u, A + M:R, :] = jnp.zeros((R - A - M, Lo), jnp.float32)
        mp = jnp.concatenate(
            [jnp.zeros((Ho, 1, Lo), jnp.float32), m,
             jnp.zeros((Ho, Wf - Wo - 1, Lo), jnp.float32)], axis=1)
        fx_ref[u, 2 * Wf:(Ho + 2) * Wf, :] = mp.reshape(Ho * Wf, Lo)
        if 2 * Wf + 1 > A:
            fx_ref[u, A:2 * Wf + 1, :] = (
                jnp.zeros((2 * Wf + 1 - A, Lo), jnp.float32))
        if A + M > (Ho + 2) * Wf:
            fx_ref[u, (Ho + 2) * Wf:A + M, :] = (
                jnp.zeros((A + M - (Ho + 2) * Wf, Lo), jnp.float32))

    def zero_pads(f_ref, u):
        if 2 * Wf + 1 > A:
            f_ref[u, A:2 * Wf + 1, :] = (
                jnp.zeros((2 * Wf + 1 - A, Lo), jnp.float32))
        for j in range(2, Ho + 2):
            f_ref[u, j * Wf + Wo + 1:j * Wf + Wf + 1, :] = zv

    def conv(u, src_ref, widx, dst_ref, res_ref):
        sh_ref[u, ...] = src_ref[u, A - Wf - 1:A + Wf - 1 + M, :]
        acc = None
        for dy in range(3):
            for dx in (0, 1):
                t = dy * 3 + dx
                w_t = wr_ref[(widx * 9 + t) * Lo:(widx * 9 + t + 1) * Lo, :]
                if dx == 1:
                    op = src_ref[u, A + (dy - 1) * Wf:A + (dy - 1) * Wf + M, :]
                else:
                    op = sh_ref[u, dy * Wf:dy * Wf + M, :]
                d = jnp.dot(op, w_t, preferred_element_type=jnp.float32)
                acc = d if acc is None else acc + d
        sh_ref[u, ...] = src_ref[u, A - Wf + 1:A + Wf + 1 + M, :]
        for dy in range(3):
            t = dy * 3 + 2
            w_t = wr_ref[(widx * 9 + t) * Lo:(widx * 9 + t + 1) * Lo, :]
            d = jnp.dot(sh_ref[u, dy * Wf:dy * Wf + M, :], w_t,
                        preferred_element_type=jnp.float32)
            acc = acc + d
        y = _mish(acc + br_ref[widx:widx + 1, :])
        if res_ref is not None:
            y = y + res_ref[u, A:A + M, :]
        dst_ref[u, A:A + M, :] = y
        zero_pads(dst_ref, u)

    for u in range(_U):
        pack_input(u)
    for u in range(_U):
        stem_pool(u)
    for u in range(_U):
        conv(u, fx_ref, 0, fh_ref, None)
    for u in range(_U):
        conv(u, fh_ref, 1, fx_ref, fx_ref)
    for u in range(_U):
        conv(u, fx_ref, 2, fh_ref, None)
    for u in range(_U):
        conv(u, fh_ref, 3, fx_ref, fx_ref)
    for u in range(_U):
        yf = fx_ref[u, 2 * Wf:(Ho + 2) * Wf, :].reshape(Ho, Wf, Lo)
        yv = yf[:, 1:Wo + 1, :].reshape(Ho * Wo, Lo)
        out_ref[u, ...] = jax.lax.dot_general(
            i128_ref[...], yv, (((1,), (1,)), ((), ())),
            preferred_element_type=jnp.float32)


def _block_diag_w(w, ci, co):
    eye = jnp.eye(_S, dtype=w.dtype)
    wb = jnp.einsum('su,co->scuo', eye, w)
    return wb.reshape(_S * ci, _S * co)


def kernel(x, conv_w, conv_b, res1_w1, res1_b1, res1_w2, res1_b2,
           res2_w1, res2_b1, res2_w2, res2_b2):
    n, cin, h, w = x.shape
    cout = conv_w.shape[-1]
    ho, wo = h // 2, w // 2
    g = n // _S
    wf, fa, fm, fr = _flat_geom(ho, wo)
    li, lo = _S * cin, _S * cout

    xv = x.reshape(g, li, h, w)
    i32 = jnp.eye(li, dtype=jnp.float32)
    i128 = jnp.eye(lo, dtype=jnp.float32)

    wc9 = conv_w.reshape(3, 3, cin, cout)
    zblk = jnp.zeros((li, lo), jnp.float32)
    we = jnp.concatenate([
        jnp.concatenate([_block_diag_w(wc9[dy, 0], cin, cout),
                         _block_diag_w(wc9[dy, 1], cin, cout),
                         _block_diag_w(wc9[dy, 2], cin, cout), zblk], axis=0)
        for dy in range(3)], axis=0)
    wod = jnp.concatenate([
        jnp.concatenate([zblk, _block_diag_w(wc9[dy, 0], cin, cout),
                         _block_diag_w(wc9[dy, 1], cin, cout),
                         _block_diag_w(wc9[dy, 2], cin, cout)], axis=0)
        for dy in range(3)], axis=0)
    wr = jnp.concatenate(
        [jnp.concatenate([_block_diag_w(m.reshape(9, cout, cout)[t],
                                        cout, cout) for t in range(9)], axis=0)
         for m in (res1_w1, res1_w2, res2_w1, res2_w2)], axis=0)
    bc = jnp.tile(conv_b.reshape(1, cout), (1, _S))
    br = jnp.concatenate(
        [jnp.tile(v.reshape(1, cout), (1, _S))
         for v in (res1_b1, res1_b2, res2_b1, res2_b2)], axis=0)

    kern = functools.partial(_impala_kernel, H=h, W=w, Cin=cin, C=cout)
    out = pl.pallas_call(
        kern,
        grid=(g // _U,),
        in_specs=[
            pl.BlockSpec((_U, li, h, w), lambda i: (i, 0, 0, 0)),
            pl.BlockSpec((li, li), lambda i: (0, 0)),
            pl.BlockSpec((lo, lo), lambda i: (0, 0)),
            pl.BlockSpec((3 * 4 * li, lo), lambda i: (0, 0)),
            pl.BlockSpec((3 * 4 * li, lo), lambda i: (0, 0)),
            pl.BlockSpec((1, lo), lambda i: (0, 0)),
            pl.BlockSpec((4 * 9 * lo, lo), lambda i: (0, 0)),
            pl.BlockSpec((4, lo), lambda i: (0, 0)),
        ],
        out_specs=pl.BlockSpec((_U, lo, ho * wo), lambda i: (i, 0, 0)),
        out_shape=jax.ShapeDtypeStruct((g, lo, ho * wo), jnp.float32),
        scratch_shapes=[
            pltpu.VMEM((_U, (h + 2) * wf, 4 * li), jnp.float32),
            pltpu.VMEM((_U, h, wo + 1, lo), jnp.float32),
            pltpu.VMEM((_U, h + 2, wo, lo), jnp.float32),
            pltpu.VMEM((_U, fr, lo), jnp.float32),
            pltpu.VMEM((_U, fr, lo), jnp.float32),
            pltpu.VMEM((_U, 2 * wf + fm, lo), jnp.float32),
        ],
        compiler_params=pltpu.CompilerParams(
            dimension_semantics=("parallel",),
            vmem_limit_bytes=100 * 1024 * 1024),
        cost_estimate=pl.CostEstimate(
            flops=2 * n * (h * w * 9 * cin * cout
                           + 4 * ho * wo * 9 * cout * cout),
            transcendentals=n * (h * w + 4 * ho * wo) * cout,
            bytes_accessed=4 * (2 * g * li * h * wo + g * lo * ho * wo),
        ),
    )(xv, i32, i128, we, wod, bc, wr, br)
    return out.reshape(n, cout, ho, wo)
